# baseline (speedup 1.0000x reference)
import functools

import jax
import jax.numpy as jnp
from jax import lax
from jax.experimental import pallas as pl
from jax.experimental.pallas import tpu as pltpu
from jax.experimental.pallas import tpu_sc as plsc

N_STATES = 1000000
OUTPUT_DIM = 16
BATCH = 16384

_info = plsc.get_sparse_core_info()
_NC, _NS, _NL = _info.num_cores, _info.num_subcores, _info.num_lanes
_NW = _NC * _NS
_B_PER_W = BATCH // _NW
_E_PER_W = _B_PER_W * OUTPUT_DIM
_CHUNK = 128
_NCHUNK = _E_PER_W // _CHUNK

_mesh = plsc.VectorSubcoreMesh(core_axis_name="c", subcore_axis_name="s")


@functools.partial(
    pl.kernel,
    mesh=_mesh,
    out_type=jax.ShapeDtypeStruct((OUTPUT_DIM, BATCH), jnp.float32),
    scratch_types=[
        pltpu.VMEM((_B_PER_W,), jnp.int32),
        pltpu.VMEM((_E_PER_W,), jnp.int32),
        pltpu.VMEM((_E_PER_W,), jnp.float32),
        pltpu.SemaphoreType.DMA,
    ],
    compiler_params=pltpu.CompilerParams(needs_layout_passes=False),
)
def _gather_kernel(table_hbm, idx_hbm, out_t_hbm, idx_v, eidx_v, rows_v, sem):
    wid = lax.axis_index("s") * _NC + lax.axis_index("c")
    base = wid * _B_PER_W
    pltpu.sync_copy(idx_hbm.at[pl.ds(base, _B_PER_W)], idx_v)

    def _expand(g, _):
        vec = idx_v[pl.ds(g * _NL, _NL)]
        for d in range(OUTPUT_DIM):
            eidx_v[pl.ds(d * _B_PER_W + g * _NL, _NL)] = vec + d * N_STATES
        return ()

    lax.fori_loop(0, _B_PER_W // _NL, _expand, ())

    def _fire(c, _):
        for b in range(4):
            j = c * 4 + b
            pltpu.make_async_copy(
                table_hbm.at[eidx_v.at[pl.ds(j * _CHUNK, _CHUNK)]],
                rows_v.at[pl.ds(j * _CHUNK, _CHUNK)],
                sem,
            ).start()
        return ()

    lax.fori_loop(0, _NCHUNK // 4, _fire, ())

    pltpu.make_async_copy(
        table_hbm.at[pl.ds(0, _E_PER_W)], rows_v, sem
    ).wait()

    for d in range(OUTPUT_DIM):
        pltpu.sync_copy(
            rows_v.at[pl.ds(d * _B_PER_W, _B_PER_W)],
            out_t_hbm.at[d, pl.ds(base, _B_PER_W)],
        )


def kernel(preprocessed_states, table):
    idx = jnp.reshape(preprocessed_states, (BATCH,)).astype(jnp.int32)
    table1d = jnp.reshape(table.T, (N_STATES * OUTPUT_DIM,))
    out_t = _gather_kernel(table1d, idx)
    return out_t.T


# speedup vs baseline: 2.7517x; 2.7517x over previous
import functools

import jax
import jax.numpy as jnp
from jax import lax
from jax.experimental import pallas as pl
from jax.experimental.pallas import tpu as pltpu
from jax.experimental.pallas import tpu_sc as plsc

N_STATES = 1000000
OUTPUT_DIM = 16
BATCH = 16384

_info = plsc.get_sparse_core_info()
_NC, _NS, _NL = _info.num_cores, _info.num_subcores, _info.num_lanes
_NW = _NC * _NS
_B_PER_W = BATCH // _NW
_CHUNK = 128
_NCHUNK = _B_PER_W // _CHUNK

_mesh = plsc.VectorSubcoreMesh(core_axis_name="c", subcore_axis_name="s")


@functools.partial(
    pl.kernel,
    mesh=_mesh,
    out_type=jax.ShapeDtypeStruct((OUTPUT_DIM, BATCH), jnp.float32),
    scratch_types=[
        pltpu.VMEM((_NCHUNK, _CHUNK), jnp.int32),
        pltpu.VMEM((_B_PER_W, OUTPUT_DIM), jnp.float32),
        pltpu.VMEM((_B_PER_W * OUTPUT_DIM,), jnp.float32),
        pltpu.SemaphoreType.DMA,
    ],
    compiler_params=pltpu.CompilerParams(
        use_tc_tiling_on_sc=False, needs_layout_passes=False
    ),
)
def _gather_kernel(table_hbm, idx_hbm, out_t_hbm, idx_v, rows_v, outb_v, sem):
    wid = lax.axis_index("s") * _NC + lax.axis_index("c")
    base = wid * _B_PER_W
    pltpu.sync_copy(idx_hbm.at[wid], idx_v)
    copies = []
    for j in range(_NCHUNK):
        copies.append(
            pltpu.async_copy(
                table_hbm.at[idx_v.at[j]],
                rows_v.at[pl.ds(j * _CHUNK, _CHUNK)],
                sem,
            )
        )
    for c in copies:
        c.wait()

    iota = lax.iota(jnp.int32, _NL)

    def _xpose(g, _):
        rows16 = g * _NL + iota
        for d in range(OUTPUT_DIM):
            vals = plsc.load_gather(
                rows_v, [rows16, jnp.full((_NL,), d, jnp.int32)]
            )
            outb_v[pl.ds(d * _B_PER_W + g * _NL, _NL)] = vals
        return ()

    lax.fori_loop(0, _B_PER_W // _NL, _xpose, ())

    for d in range(OUTPUT_DIM):
        pltpu.sync_copy(
            outb_v.at[pl.ds(d * _B_PER_W, _B_PER_W)],
            out_t_hbm.at[d, pl.ds(base, _B_PER_W)],
        )


def kernel(preprocessed_states, table):
    idx = jnp.reshape(preprocessed_states, (_NW, _NCHUNK, _CHUNK)).astype(
        jnp.int32
    )
    out_t = _gather_kernel(table, idx)
    return out_t.T


# speedup vs baseline: 2.7521x; 1.0001x over previous
import functools

import jax
import jax.numpy as jnp
from jax import lax
from jax.experimental import pallas as pl
from jax.experimental.pallas import tpu as pltpu
from jax.experimental.pallas import tpu_sc as plsc

N_STATES = 1000000
OUTPUT_DIM = 16
BATCH = 16384

_info = plsc.get_sparse_core_info()
_NC, _NS, _NL = _info.num_cores, _info.num_subcores, _info.num_lanes
_NW = _NC * _NS
_B_PER_W = BATCH // _NW
_CHUNK = 128
_NCHUNK = _B_PER_W // _CHUNK

_mesh = plsc.VectorSubcoreMesh(core_axis_name="c", subcore_axis_name="s")


@functools.partial(
    pl.kernel,
    mesh=_mesh,
    out_type=jax.ShapeDtypeStruct((OUTPUT_DIM, BATCH), jnp.float32),
    scratch_types=[
        pltpu.VMEM((_B_PER_W,), jnp.int32),
        pltpu.VMEM((_B_PER_W, OUTPUT_DIM), jnp.float32),
        pltpu.VMEM((_B_PER_W * OUTPUT_DIM,), jnp.float32),
        pltpu.SemaphoreType.DMA,
    ],
    compiler_params=pltpu.CompilerParams(
        use_tc_tiling_on_sc=False, needs_layout_passes=False
    ),
)
def _gather_kernel(table_hbm, idx_hbm, out_t_hbm, idx_v, rows_v, outb_v, sem):
    wid = lax.axis_index("s") * _NC + lax.axis_index("c")
    base = wid * _B_PER_W
    pltpu.sync_copy(idx_hbm.at[pl.ds(base, _B_PER_W)], idx_v)
    copies = []
    for j in range(_NCHUNK):
        copies.append(
            pltpu.async_copy(
                table_hbm.at[idx_v.at[pl.ds(j * _CHUNK, _CHUNK)]],
                rows_v.at[pl.ds(j * _CHUNK, _CHUNK)],
                sem,
            )
        )
    for c in copies:
        c.wait()

    iota = lax.iota(jnp.int32, _NL)

    def _xpose(g, _):
        rows16 = g * _NL + iota
        for d in range(OUTPUT_DIM):
            vals = plsc.load_gather(
                rows_v, [rows16, jnp.full((_NL,), d, jnp.int32)]
            )
            outb_v[pl.ds(d * _B_PER_W + g * _NL, _NL)] = vals
        return ()

    lax.fori_loop(0, _B_PER_W // _NL, _xpose, ())

    for d in range(OUTPUT_DIM):
        pltpu.sync_copy(
            outb_v.at[pl.ds(d * _B_PER_W, _B_PER_W)],
            out_t_hbm.at[d, pl.ds(base, _B_PER_W)],
        )


def kernel(preprocessed_states, table):
    idx = jnp.reshape(preprocessed_states, (BATCH,)).astype(jnp.int32)
    out_t = _gather_kernel(table, idx)
    return out_t.T


# speedup vs baseline: 4.5366x; 1.6484x over previous
import functools

import jax
import jax.numpy as jnp
from jax import lax
from jax.experimental import pallas as pl
from jax.experimental.pallas import tpu as pltpu
from jax.experimental.pallas import tpu_sc as plsc

N_STATES = 1000000
OUTPUT_DIM = 16
BATCH = 16384

_info = plsc.get_sparse_core_info()
_NC, _NS = _info.num_cores, _info.num_subcores
_NW = _NC * _NS
_B_PER_W = BATCH // _NW
_UNROLL = 16
_NSTEP = _B_PER_W // _UNROLL

_mesh = plsc.VectorSubcoreMesh(core_axis_name="c", subcore_axis_name="s")


@functools.partial(
    pl.kernel,
    mesh=_mesh,
    out_type=jax.ShapeDtypeStruct((BATCH, OUTPUT_DIM), jnp.float32),
    scratch_types=[
        pltpu.VMEM((_B_PER_W,), jnp.int32),
        pltpu.VMEM((_B_PER_W, OUTPUT_DIM), jnp.float32),
        pltpu.SemaphoreType.DMA,
    ],
)
def _gather_kernel(table_hbm, idx_hbm, out_hbm, idx_v, rows_v, sem):
    wid = lax.axis_index("s") * _NC + lax.axis_index("c")
    base = wid * _B_PER_W
    pltpu.sync_copy(idx_hbm.at[pl.ds(base, _B_PER_W)], idx_v)

    def _start(g, _):
        vec = idx_v[pl.ds(g * _UNROLL, _UNROLL)]
        for b in range(_UNROLL):
            i = g * _UNROLL + b
            r = vec[b]
            pltpu.make_async_copy(
                table_hbm.at[pl.ds(r, 1), :],
                rows_v.at[pl.ds(i, 1), :],
                sem,
            ).start()
        return ()

    lax.fori_loop(0, _NSTEP, _start, ())

    pltpu.make_async_copy(
        table_hbm.at[pl.ds(0, _B_PER_W), :], rows_v, sem
    ).wait()

    pltpu.sync_copy(rows_v, out_hbm.at[pl.ds(base, _B_PER_W), :])


def kernel(preprocessed_states, table):
    idx = jnp.reshape(preprocessed_states, (BATCH,)).astype(jnp.int32)
    return _gather_kernel(table, idx)
